# Initial kernel scaffold; baseline (speedup 1.0000x reference)
import jax
import jax.numpy as jnp
from jax.experimental import pallas as pl


def kernel(x, edge_index, kpos1, kpos2, W1, W2, g1, b1, g2, b2):
    raise NotImplementedError("write your pallas kernel here")


# speedup vs baseline: 3.8062x; 3.8062x over previous
import functools

import jax
import jax.numpy as jnp
from jax import lax
from jax.experimental import pallas as pl
from jax.experimental.pallas import tpu as pltpu
from jax.experimental.pallas import tpu_sc as plsc

_N = 10000
_E = 320000
_C = 128
_K = 27

_NC = 2
_NS = 16
_NW = _NC * _NS
_EPW = _E // _NW
_CH = 80
_NCH = _EPW // _CH
_NP = 10112
_RPS = _NP // _NS


def _sc_scatter_kernel(z_hbm, src_hbm, kpos_hbm, dst2_hbm, zeros_hbm, out_hbm,
                       gidx_v, kpos_v, dst_v, rows_v, acc, sem):
    c = lax.axis_index("c")
    s = lax.axis_index("s")
    wid = s * _NC + c
    base = wid * _EPW

    pltpu.sync_copy(zeros_hbm, acc.at[pl.ds(s * _RPS, _RPS)])

    pltpu.sync_copy(src_hbm.at[pl.ds(base, _EPW)], gidx_v)
    pltpu.sync_copy(kpos_hbm.at[pl.ds(base, _EPW)], kpos_v)
    pltpu.sync_copy(dst2_hbm.at[wid], dst_v)

    def _gidx_body(i, _):
        off = pl.multiple_of(i * 16, 16)
        sv = gidx_v[pl.ds(off, 16)]
        kv = kpos_v[pl.ds(off, 16)]
        gidx_v[pl.ds(off, 16)] = sv * _K + kv
        return 0

    lax.fori_loop(0, _EPW // 16, _gidx_body, 0)

    plsc.subcore_barrier()

    def _chunk_body(j, _):
        idx = gidx_v.at[pl.ds(j * _CH, _CH)]
        pltpu.async_copy(z_hbm.at[idx], rows_v, sem).wait()
        pltpu.sync_copy(rows_v, acc.at[dst_v.at[j]], add=True)
        return 0

    lax.fori_loop(0, _NCH, _chunk_body, 0)

    plsc.subcore_barrier()

    pltpu.sync_copy(acc.at[pl.ds(s * _RPS, _RPS)],
                    out_hbm.at[c, pl.ds(s * _RPS, _RPS)])


def _sc_scatter(z2d, src, kpos, dst2, zeros):
    mesh = plsc.VectorSubcoreMesh(core_axis_name="c", subcore_axis_name="s")
    f = functools.partial(
        pl.kernel,
        mesh=mesh,
        out_type=jax.ShapeDtypeStruct((_NC, _NP, _C), jnp.float32),
        scratch_types=[
            pltpu.VMEM((_EPW,), jnp.int32),
            pltpu.VMEM((_EPW,), jnp.int32),
            pltpu.VMEM((_NCH, _CH), jnp.int32),
            pltpu.VMEM((_CH, _C), jnp.float32),
            pltpu.VMEM_SHARED((_NP, _C), jnp.float32),
            pltpu.SemaphoreType.DMA,
        ],
    )(_sc_scatter_kernel)
    return f(z2d, src, kpos, dst2, zeros)


def _mm_body(a_ref, b_ref, o_ref):
    o_ref[...] = jnp.dot(a_ref[...], b_ref[...],
                         preferred_element_type=jnp.float32,
                         precision=lax.Precision.HIGHEST)


def _matmul(a, b):
    bm = 400
    return pl.pallas_call(
        _mm_body,
        grid=(_N // bm,),
        in_specs=[
            pl.BlockSpec((bm, _C), lambda i: (i, 0)),
            pl.BlockSpec((_C, _K * _C), lambda i: (0, 0)),
        ],
        out_specs=pl.BlockSpec((bm, _K * _C), lambda i: (i, 0)),
        out_shape=jax.ShapeDtypeStruct((_N, _K * _C), jnp.float32),
    )(a, b)


def _bn_relu_body(p_ref, g_ref, b_ref, o_ref):
    p = p_ref[...]
    u = p[0, :_N] + p[1, :_N]
    mu = jnp.mean(u, axis=0, keepdims=True)
    d = u - mu
    var = jnp.mean(d * d, axis=0, keepdims=True)
    y = g_ref[...] * d * lax.rsqrt(var + 1e-5) + b_ref[...]
    o_ref[...] = jnp.maximum(y, 0.0)


def _bn_relu(p, g, b):
    return pl.pallas_call(
        _bn_relu_body,
        out_shape=jax.ShapeDtypeStruct((_N, _C), jnp.float32),
    )(p, g.reshape(1, _C), b.reshape(1, _C))


def _bn_res_relu_body(p_ref, g_ref, b_ref, x_ref, o_ref):
    p = p_ref[...]
    u = p[0, :_N] + p[1, :_N]
    mu = jnp.mean(u, axis=0, keepdims=True)
    d = u - mu
    var = jnp.mean(d * d, axis=0, keepdims=True)
    y = g_ref[...] * d * lax.rsqrt(var + 1e-5) + b_ref[...]
    o_ref[...] = jnp.maximum(y + x_ref[...], 0.0)


def _bn_res_relu(p, g, b, x):
    return pl.pallas_call(
        _bn_res_relu_body,
        out_shape=jax.ShapeDtypeStruct((_N, _C), jnp.float32),
    )(p, g.reshape(1, _C), b.reshape(1, _C), x)


def kernel(x, edge_index, kpos1, kpos2, W1, W2, g1, b1, g2, b2):
    src = edge_index[0]
    dst2 = edge_index[1].reshape(_NW, _NCH, _CH)
    zeros = jnp.zeros((_RPS, _C), jnp.float32)
    W1r = jnp.transpose(W1, (1, 0, 2)).reshape(_C, _K * _C)
    W2r = jnp.transpose(W2, (1, 0, 2)).reshape(_C, _K * _C)

    z1 = _matmul(x, W1r)
    p1 = _sc_scatter(z1.reshape(_N * _K, _C), src, kpos1, dst2, zeros)
    u1 = _bn_relu(p1, g1, b1)
    z2 = _matmul(u1, W2r)
    p2 = _sc_scatter(z2.reshape(_N * _K, _C), src, kpos2, dst2, zeros)
    return _bn_res_relu(p2, g2, b2, x)


# speedup vs baseline: 4.2860x; 1.1261x over previous
import functools

import jax
import jax.numpy as jnp
from jax import lax
from jax.experimental import pallas as pl
from jax.experimental.pallas import tpu as pltpu
from jax.experimental.pallas import tpu_sc as plsc

_N = 10000
_E = 320000
_C = 128
_K = 27

_NC = 2
_NS = 16
_NW = _NC * _NS
_EPW = _E // _NW
_CH = 80
_NCH = _EPW // _CH
_NP = 10112
_RPS = _NP // _NS


def _sc_scatter_kernel(z_hbm, src_hbm, kpos3_hbm, dst3_hbm, zeros_hbm, out_hbm,
                       gidx_v, dst_v, rows0_v, rows1_v, acc, sem0, sem1):
    c = lax.axis_index("c")
    s = lax.axis_index("s")
    wid = s * _NC + c
    base = wid * _EPW

    pltpu.sync_copy(zeros_hbm, acc.at[pl.ds(s * _RPS, _RPS)])

    pltpu.sync_copy(src_hbm.at[pl.ds(base, _EPW)], gidx_v)
    pltpu.sync_copy(kpos3_hbm.at[wid], dst_v)

    def _gidx_body(r, _):
        for cc in range(_CH // 16):
            off = pl.multiple_of(r * _CH + cc * 16, 16)
            sv = gidx_v[pl.ds(off, 16)]
            kv = dst_v[r, pl.ds(cc * 16, 16)]
            gidx_v[pl.ds(off, 16)] = kv * _N + sv
        return 0

    lax.fori_loop(0, _NCH, _gidx_body, 0)

    pltpu.sync_copy(dst3_hbm.at[wid], dst_v)

    plsc.subcore_barrier()

    def _gather(j, rows, sem):
        return pltpu.async_copy(z_hbm.at[gidx_v.at[pl.ds(j * _CH, _CH)]],
                                rows, sem)

    _gather(0, rows0_v, sem0)

    def _chunk_body(jj, _):
        j0 = jj * 2
        _gather(j0 + 1, rows1_v, sem1)
        pltpu.make_async_copy(z_hbm.at[pl.ds(0, _CH)], rows0_v, sem0).wait()
        pltpu.sync_copy(rows0_v, acc.at[dst_v.at[j0]], add=True)
        _gather(j0 + 2, rows0_v, sem0)
        pltpu.make_async_copy(z_hbm.at[pl.ds(0, _CH)], rows1_v, sem1).wait()
        pltpu.sync_copy(rows1_v, acc.at[dst_v.at[j0 + 1]], add=True)
        return 0

    lax.fori_loop(0, (_NCH - 1) // 2, _chunk_body, 0)

    pltpu.make_async_copy(z_hbm.at[pl.ds(0, _CH)], rows0_v, sem0).wait()
    pltpu.sync_copy(rows0_v, acc.at[dst_v.at[_NCH - 1]], add=True)

    plsc.subcore_barrier()

    pltpu.sync_copy(acc.at[pl.ds(s * _RPS, _RPS)],
                    out_hbm.at[c, pl.ds(s * _RPS, _RPS)])


def _sc_scatter(z2d, src, kpos3, dst3, zeros):
    mesh = plsc.VectorSubcoreMesh(core_axis_name="c", subcore_axis_name="s")
    f = functools.partial(
        pl.kernel,
        mesh=mesh,
        out_type=jax.ShapeDtypeStruct((_NC, _NP, _C), jnp.float32),
        scratch_types=[
            pltpu.VMEM((_EPW,), jnp.int32),
            pltpu.VMEM((_NCH, _CH), jnp.int32),
            pltpu.VMEM((_CH, _C), jnp.float32),
            pltpu.VMEM((_CH, _C), jnp.float32),
            pltpu.VMEM_SHARED((_NP, _C), jnp.float32),
            pltpu.SemaphoreType.DMA,
            pltpu.SemaphoreType.DMA,
        ],
    )(_sc_scatter_kernel)
    return f(z2d, src, kpos3, dst3, zeros)


def _mm_body(a_ref, w_ref, o_ref):
    o_ref[...] = jnp.dot(a_ref[...], w_ref[0],
                         preferred_element_type=jnp.float32,
                         precision=lax.Precision.HIGHEST)


def _matmul(a, w):
    bm = 1000
    nb = _N // bm
    return pl.pallas_call(
        _mm_body,
        grid=(nb, _K),
        in_specs=[
            pl.BlockSpec((bm, _C), lambda i, k: (i, 0)),
            pl.BlockSpec((1, _C, _C), lambda i, k: (k, 0, 0)),
        ],
        out_specs=pl.BlockSpec((bm, _C), lambda i, k: (k * nb + i, 0)),
        out_shape=jax.ShapeDtypeStruct((_K * _N, _C), jnp.float32),
    )(a, w)


def _bn_relu_body(p_ref, g_ref, b_ref, o_ref):
    p = p_ref[...]
    u = p[0, :_N] + p[1, :_N]
    mu = jnp.mean(u, axis=0, keepdims=True)
    d = u - mu
    var = jnp.mean(d * d, axis=0, keepdims=True)
    y = g_ref[...] * d * lax.rsqrt(var + 1e-5) + b_ref[...]
    o_ref[...] = jnp.maximum(y, 0.0)


def _bn_relu(p, g, b):
    return pl.pallas_call(
        _bn_relu_body,
        out_shape=jax.ShapeDtypeStruct((_N, _C), jnp.float32),
    )(p, g.reshape(1, _C), b.reshape(1, _C))


def _bn_res_relu_body(p_ref, g_ref, b_ref, x_ref, o_ref):
    p = p_ref[...]
    u = p[0, :_N] + p[1, :_N]
    mu = jnp.mean(u, axis=0, keepdims=True)
    d = u - mu
    var = jnp.mean(d * d, axis=0, keepdims=True)
    y = g_ref[...] * d * lax.rsqrt(var + 1e-5) + b_ref[...]
    o_ref[...] = jnp.maximum(y + x_ref[...], 0.0)


def _bn_res_relu(p, g, b, x):
    return pl.pallas_call(
        _bn_res_relu_body,
        out_shape=jax.ShapeDtypeStruct((_N, _C), jnp.float32),
    )(p, g.reshape(1, _C), b.reshape(1, _C), x)


def kernel(x, edge_index, kpos1, kpos2, W1, W2, g1, b1, g2, b2):
    src = edge_index[0]
    dst3 = edge_index[1].reshape(_NW, _NCH, _CH)
    kp1 = kpos1.reshape(_NW, _NCH, _CH)
    kp2 = kpos2.reshape(_NW, _NCH, _CH)
    zeros = jnp.zeros((_RPS, _C), jnp.float32)

    z1 = _matmul(x, W1)
    p1 = _sc_scatter(z1, src, kp1, dst3, zeros)
    u1 = _bn_relu(p1, g1, b1)
    z2 = _matmul(u1, W2)
    p2 = _sc_scatter(z2, src, kp2, dst3, zeros)
    return _bn_res_relu(p2, g2, b2, x)


# speedup vs baseline: 5.3660x; 1.2520x over previous
import functools

import jax
import jax.numpy as jnp
from jax import lax
from jax.experimental import pallas as pl
from jax.experimental.pallas import tpu as pltpu
from jax.experimental.pallas import tpu_sc as plsc

_N = 10000
_E = 320000
_C = 128
_K = 27

_NC = 2
_NS = 16
_NW = _NC * _NS
_EPW = _E // _NW
_CH = 80
_NCH = _EPW // _CH
_NP = 10112
_RPS = _NP // _NS


def _sc_scatter_kernel(z_hbm, src_hbm, kpos3_hbm, dst3_hbm, zeros_hbm, out_hbm,
                       gidx_v, dst_v, rows0_v, rows1_v, acc, sem0, sem1):
    c = lax.axis_index("c")
    s = lax.axis_index("s")
    wid = s * _NC + c
    base = wid * _EPW

    pltpu.sync_copy(zeros_hbm, acc.at[pl.ds(s * _RPS, _RPS)])

    pltpu.sync_copy(src_hbm.at[pl.ds(base, _EPW)], gidx_v)
    pltpu.sync_copy(kpos3_hbm.at[wid], dst_v)

    def _gidx_body(r, _):
        for cc in range(_CH // 16):
            off = pl.multiple_of(r * _CH + cc * 16, 16)
            sv = gidx_v[pl.ds(off, 16)]
            kv = dst_v[r, pl.ds(cc * 16, 16)]
            gidx_v[pl.ds(off, 16)] = kv * _N + sv
        return 0

    lax.fori_loop(0, _NCH, _gidx_body, 0)

    pltpu.sync_copy(dst3_hbm.at[wid], dst_v)

    plsc.subcore_barrier()

    def _gather(j, rows, sem):
        return pltpu.async_copy(z_hbm.at[gidx_v.at[pl.ds(j * _CH, _CH)]],
                                rows, sem)

    _gather(0, rows0_v, sem0)

    def _chunk_body(jj, _):
        j0 = jj * 2
        _gather(j0 + 1, rows1_v, sem1)
        pltpu.make_async_copy(z_hbm.at[pl.ds(0, _CH)], rows0_v, sem0).wait()
        pltpu.sync_copy(rows0_v, acc.at[dst_v.at[j0]], add=True)
        _gather(j0 + 2, rows0_v, sem0)
        pltpu.make_async_copy(z_hbm.at[pl.ds(0, _CH)], rows1_v, sem1).wait()
        pltpu.sync_copy(rows1_v, acc.at[dst_v.at[j0 + 1]], add=True)
        return 0

    lax.fori_loop(0, (_NCH - 1) // 2, _chunk_body, 0)

    pltpu.make_async_copy(z_hbm.at[pl.ds(0, _CH)], rows0_v, sem0).wait()
    pltpu.sync_copy(rows0_v, acc.at[dst_v.at[_NCH - 1]], add=True)

    plsc.subcore_barrier()

    pltpu.sync_copy(acc.at[pl.ds(s * _RPS, _RPS)],
                    out_hbm.at[c, pl.ds(s * _RPS, _RPS)])


def _sc_scatter(z2d, src, kpos3, dst3, zeros):
    mesh = plsc.VectorSubcoreMesh(core_axis_name="c", subcore_axis_name="s")
    f = functools.partial(
        pl.kernel,
        mesh=mesh,
        out_type=jax.ShapeDtypeStruct((_NC, _NP, _C), jnp.float32),
        scratch_types=[
            pltpu.VMEM((_EPW,), jnp.int32),
            pltpu.VMEM((_NCH, _CH), jnp.int32),
            pltpu.VMEM((_CH, _C), jnp.float32),
            pltpu.VMEM((_CH, _C), jnp.float32),
            pltpu.VMEM_SHARED((_NP, _C), jnp.float32),
            pltpu.SemaphoreType.DMA,
            pltpu.SemaphoreType.DMA,
        ],
    )(_sc_scatter_kernel)
    return f(z2d, src, kpos3, dst3, zeros)


def _mm_body(a_ref, w_ref, o_ref):
    o_ref[...] = jnp.dot(a_ref[...], w_ref[0],
                         preferred_element_type=jnp.float32,
                         precision=lax.Precision.DEFAULT)


def _matmul(a, w):
    bm = 1000
    nb = _N // bm
    return pl.pallas_call(
        _mm_body,
        grid=(nb, _K),
        in_specs=[
            pl.BlockSpec((bm, _C), lambda i, k: (i, 0)),
            pl.BlockSpec((1, _C, _C), lambda i, k: (k, 0, 0)),
        ],
        out_specs=pl.BlockSpec((bm, _C), lambda i, k: (k * nb + i, 0)),
        out_shape=jax.ShapeDtypeStruct((_K * _N, _C), jnp.float32),
    )(a, w)


def _bn_relu_body(p_ref, g_ref, b_ref, o_ref):
    p = p_ref[...]
    u = p[0, :_N] + p[1, :_N]
    mu = jnp.mean(u, axis=0, keepdims=True)
    d = u - mu
    var = jnp.mean(d * d, axis=0, keepdims=True)
    y = g_ref[...] * d * lax.rsqrt(var + 1e-5) + b_ref[...]
    o_ref[...] = jnp.maximum(y, 0.0)


def _bn_relu(p, g, b):
    return pl.pallas_call(
        _bn_relu_body,
        out_shape=jax.ShapeDtypeStruct((_N, _C), jnp.float32),
    )(p, g.reshape(1, _C), b.reshape(1, _C))


def _bn_res_relu_body(p_ref, g_ref, b_ref, x_ref, o_ref):
    p = p_ref[...]
    u = p[0, :_N] + p[1, :_N]
    mu = jnp.mean(u, axis=0, keepdims=True)
    d = u - mu
    var = jnp.mean(d * d, axis=0, keepdims=True)
    y = g_ref[...] * d * lax.rsqrt(var + 1e-5) + b_ref[...]
    o_ref[...] = jnp.maximum(y + x_ref[...], 0.0)


def _bn_res_relu(p, g, b, x):
    return pl.pallas_call(
        _bn_res_relu_body,
        out_shape=jax.ShapeDtypeStruct((_N, _C), jnp.float32),
    )(p, g.reshape(1, _C), b.reshape(1, _C), x)


def kernel(x, edge_index, kpos1, kpos2, W1, W2, g1, b1, g2, b2):
    src = edge_index[0]
    dst3 = edge_index[1].reshape(_NW, _NCH, _CH)
    kp1 = kpos1.reshape(_NW, _NCH, _CH)
    kp2 = kpos2.reshape(_NW, _NCH, _CH)
    zeros = jnp.zeros((_RPS, _C), jnp.float32)

    z1 = _matmul(x, W1)
    p1 = _sc_scatter(z1, src, kp1, dst3, zeros)
    u1 = _bn_relu(p1, g1, b1)
    z2 = _matmul(u1, W2)
    p2 = _sc_scatter(z2, src, kp2, dst3, zeros)
    return _bn_res_relu(p2, g2, b2, x)
